# baseline (speedup 1.0000x reference)
import functools

import jax
import jax.numpy as jnp
from jax import lax
from jax.experimental import pallas as pl
from jax.experimental.pallas import tpu as pltpu
from jax.experimental.pallas import tpu_sc as plsc

MAX_LEN = 2048
NUM_HEADS = 16
PAD_W = 2 * MAX_LEN
NSHIFT = 8
GROUP = 8
GSEM = 4

_info = plsc.get_sparse_core_info()
_NC, _NS = _info.num_cores, _info.num_subcores
_NW = _NC * _NS
_ROWS_PER = (NUM_HEADS * MAX_LEN) // _NW
_WPH = MAX_LEN // _ROWS_PER


def _make_sc_kernel():
    mesh = plsc.VectorSubcoreMesh(core_axis_name="c", subcore_axis_name="s")

    @functools.partial(
        pl.kernel,
        mesh=mesh,
        out_type=jax.ShapeDtypeStruct((NUM_HEADS * MAX_LEN * MAX_LEN,), jnp.float32),
        scratch_types=[pltpu.VMEM((PAD_W,), jnp.float32)] * NSHIFT
        + [pltpu.VMEM((GROUP * MAX_LEN,), jnp.float32)]
        + [pltpu.SemaphoreType.DMA] * GSEM,
    )
    def sc_bias(tt8_hbm, out_hbm, *scratch):
        vs = scratch[:NSHIFT]
        drain_v = scratch[NSHIFT]
        sems = scratch[NSHIFT + 1 :]
        wid = lax.axis_index("s") * _NC + lax.axis_index("c")
        h = wid // _WPH
        i0 = (wid % _WPH) * _ROWS_PER

        for s in range(NSHIFT):
            pltpu.sync_copy(tt8_hbm.at[pl.ds((h * NSHIFT + s) * PAD_W, PAD_W)], vs[s])

        kmax = _ROWS_PER // NSHIFT
        ngrp = kmax // GROUP

        def group_wait(b):
            pltpu.make_async_copy(
                drain_v, out_hbm.at[pl.ds(0, GROUP * MAX_LEN)], sems[b]
            ).wait()

        for r in range(NSHIFT):
            s_r = (MAX_LEN - 1 - r) % NSHIFT
            base = MAX_LEN - 1 - s_r - r - i0

            def blk(g, carry, r=r, s_r=s_r, base=base):
                for b in range(GSEM):
                    @pl.when(g > 0)
                    def _wait():
                        group_wait(b)

                    for q in range(GROUP):
                        k = (g * GSEM + b) * GROUP + q
                        i = i0 + r + NSHIFT * k
                        off = pl.multiple_of(base - NSHIFT * k, NSHIFT)
                        src = vs[s_r].at[pl.ds(off, MAX_LEN)]
                        dst = out_hbm.at[pl.ds((h * MAX_LEN + i) * MAX_LEN, MAX_LEN)]
                        pltpu.make_async_copy(src, dst, sems[b]).start()
                return carry

            lax.fori_loop(0, ngrp // GSEM, blk, 0)

            for b in range(GSEM):
                group_wait(b)

    return sc_bias


_sc_bias = _make_sc_kernel()


@jax.jit
def kernel(T, table):
    del T
    ttp = jnp.pad(jnp.transpose(table), ((0, 0), (0, NSHIFT + 1)))
    tt8 = jnp.stack(
        [ttp[:, s : s + PAD_W] for s in range(NSHIFT)], axis=1
    )
    out = _sc_bias(tt8.reshape(-1))
    return out.reshape(NUM_HEADS, MAX_LEN, MAX_LEN)


# speedup vs baseline: 1.0054x; 1.0054x over previous
import functools

import jax
import jax.numpy as jnp
from jax import lax
from jax.experimental import pallas as pl
from jax.experimental.pallas import tpu as pltpu
from jax.experimental.pallas import tpu_sc as plsc

MAX_LEN = 2048
NUM_HEADS = 16
PAD_W = 2 * MAX_LEN
NSHIFT = 8
GROUP = 8
GSEM = 4

_info = plsc.get_sparse_core_info()
_NC, _NS = _info.num_cores, _info.num_subcores
_NW = _NC * _NS
_ROWS_PER = (NUM_HEADS * MAX_LEN) // _NW
_WPH = MAX_LEN // _ROWS_PER


def _make_sc_kernel():
    mesh = plsc.VectorSubcoreMesh(core_axis_name="c", subcore_axis_name="s")

    @functools.partial(
        pl.kernel,
        mesh=mesh,
        out_type=jax.ShapeDtypeStruct((NUM_HEADS * MAX_LEN * MAX_LEN,), jnp.float32),
        scratch_types=[pltpu.VMEM((PAD_W,), jnp.float32)] * NSHIFT
        + [pltpu.VMEM((GROUP * MAX_LEN,), jnp.float32)]
        + [pltpu.SemaphoreType.DMA] * GSEM,
    )
    def sc_bias(tt8_hbm, out_hbm, *scratch):
        vs = scratch[:NSHIFT]
        drain_v = scratch[NSHIFT]
        sems = scratch[NSHIFT + 1 :]
        wid = lax.axis_index("s") * _NC + lax.axis_index("c")
        h = wid // _WPH
        i0 = (wid % _WPH) * _ROWS_PER

        for s in range(NSHIFT):
            pltpu.sync_copy(tt8_hbm.at[pl.ds((h * NSHIFT + s) * PAD_W, PAD_W)], vs[s])

        kmax = _ROWS_PER // NSHIFT
        ngrp = kmax // GROUP

        def group_wait(b):
            pltpu.make_async_copy(
                drain_v, out_hbm.at[pl.ds(0, GROUP * MAX_LEN)], sems[b]
            ).wait()

        def blk(g, carry):
            for b in range(GSEM):
                @pl.when(g > 0)
                def _wait():
                    group_wait(b)

                k = g * GSEM + b
                for r in range(NSHIFT):
                    s_r = (MAX_LEN - 1 - r) % NSHIFT
                    base = MAX_LEN - 1 - s_r - r - i0
                    i = i0 + NSHIFT * k + r
                    off = pl.multiple_of(base - NSHIFT * k, NSHIFT)
                    src = vs[s_r].at[pl.ds(off, MAX_LEN)]
                    dst = out_hbm.at[pl.ds((h * MAX_LEN + i) * MAX_LEN, MAX_LEN)]
                    pltpu.make_async_copy(src, dst, sems[b]).start()
            return carry

        lax.fori_loop(0, (_ROWS_PER // NSHIFT) // GSEM, blk, 0)

        for b in range(GSEM):
            group_wait(b)

    return sc_bias


_sc_bias = _make_sc_kernel()


@jax.jit
def kernel(T, table):
    del T
    ttp = jnp.pad(jnp.transpose(table), ((0, 0), (0, NSHIFT + 1)))
    tt8 = jnp.stack(
        [ttp[:, s : s + PAD_W] for s in range(NSHIFT)], axis=1
    )
    out = _sc_bias(tt8.reshape(-1))
    return out.reshape(NUM_HEADS, MAX_LEN, MAX_LEN)
